# baseline (speedup 1.0000x reference)
import jax
import jax.numpy as jnp
from jax.experimental import pallas as pl
from jax.experimental.pallas import tpu as pltpu
from jax.sharding import Mesh, NamedSharding, PartitionSpec as P

_OUT_DIM = 200
_BLOCK_B = 8192


def _mlp_tile_kernel(x_ref, w1_ref, b1_ref, w2_ref, b2_ref, out_ref):
    h = jnp.dot(x_ref[...], w1_ref[...], preferred_element_type=jnp.float32)
    h = jnp.maximum(h + b1_ref[...], 0.0)
    q = jnp.dot(h, w2_ref[...], preferred_element_type=jnp.float32)
    out_ref[...] = q + b2_ref[...]


def _forward_one_core(x, w1_t, b1, w2_t, b2):
    B, S = x.shape
    Hp = w1_t.shape[1]
    Op = w2_t.shape[1]
    y = pl.pallas_call(
        _mlp_tile_kernel,
        out_shape=jax.ShapeDtypeStruct((B, _OUT_DIM), jnp.float32),
        grid=(B // _BLOCK_B,),
        in_specs=[
            pl.BlockSpec((_BLOCK_B, S), lambda i: (i, 0)),
            pl.BlockSpec((S, Hp), lambda i: (0, 0)),
            pl.BlockSpec((1, Hp), lambda i: (0, 0)),
            pl.BlockSpec((Hp, _OUT_DIM), lambda i: (0, 0)),
            pl.BlockSpec((1, _OUT_DIM), lambda i: (0, 0)),
        ],
        out_specs=pl.BlockSpec((_BLOCK_B, _OUT_DIM), lambda i: (i, 0)),
        compiler_params=pltpu.CompilerParams(
            dimension_semantics=("parallel",),
        ),
        cost_estimate=pl.CostEstimate(
            flops=2 * B * (S * Hp + Hp * Op),
            transcendentals=0,
            bytes_accessed=4 * (B * S + S * Hp + Hp + Hp * Op + Op + B * Op),
        ),
    )(x, w1_t, b1, w2_t[:, :_OUT_DIM], b2[:, :_OUT_DIM])
    return y


@jax.jit
def kernel(x, w1_t, b1, w2_t, b2):
    devs = jax.devices()
    if len(devs) < 2:
        return _forward_one_core(x, w1_t, b1, w2_t, b2)
    mesh = Mesh(devs[:2], ("b",))
    shard = NamedSharding(mesh, P("b", None))
    repl = NamedSharding(mesh, P(None, None))
    x = jax.lax.with_sharding_constraint(x, shard)
    w1_t = jax.lax.with_sharding_constraint(w1_t, repl)
    b1 = jax.lax.with_sharding_constraint(b1, repl)
    w2_t = jax.lax.with_sharding_constraint(w2_t, repl)
    b2 = jax.lax.with_sharding_constraint(b2, repl)
    fwd = jax.shard_map(
        _forward_one_core, mesh=mesh,
        in_specs=(P("b", None), P(None, None), P(None, None),
                  P(None, None), P(None, None)),
        out_specs=P("b", None), check_vma=False,
    )
    return fwd(x, w1_t, b1, w2_t, b2)


# speedup vs baseline: 1.1142x; 1.1142x over previous
import jax
import jax.numpy as jnp
from jax.experimental import pallas as pl
from jax.experimental.pallas import tpu as pltpu
from jax.sharding import Mesh, NamedSharding, PartitionSpec as P

_OUT_DIM = 200
_BLOCK_B = 16384


def _mlp_tile_kernel(x_ref, w1_ref, b1_ref, w2_ref, b2_ref, out_ref):
    h = jnp.dot(x_ref[...], w1_ref[...], preferred_element_type=jnp.float32)
    h = jnp.maximum(h + b1_ref[...], 0.0)
    q = jnp.dot(h, w2_ref[...], preferred_element_type=jnp.float32)
    out_ref[...] = q + b2_ref[...]


def _forward_one_core(x, w1_t, b1, w2_t, b2):
    B, S = x.shape
    Hp = w1_t.shape[1]
    Op = w2_t.shape[1]
    y = pl.pallas_call(
        _mlp_tile_kernel,
        out_shape=jax.ShapeDtypeStruct((B, Op), jnp.float32),
        grid=(B // _BLOCK_B,),
        in_specs=[
            pl.BlockSpec((_BLOCK_B, S), lambda i: (i, 0)),
            pl.BlockSpec((S, Hp), lambda i: (0, 0)),
            pl.BlockSpec((1, Hp), lambda i: (0, 0)),
            pl.BlockSpec((Hp, Op), lambda i: (0, 0)),
            pl.BlockSpec((1, Op), lambda i: (0, 0)),
        ],
        out_specs=pl.BlockSpec((_BLOCK_B, Op), lambda i: (i, 0)),
        compiler_params=pltpu.CompilerParams(
            dimension_semantics=("parallel",),
        ),
        cost_estimate=pl.CostEstimate(
            flops=2 * B * (S * Hp + Hp * Op),
            transcendentals=0,
            bytes_accessed=4 * (B * S + S * Hp + Hp + Hp * Op + Op + B * Op),
        ),
    )(x, w1_t, b1, w2_t, b2)
    return y[:, :_OUT_DIM]


@jax.jit
def kernel(x, w1_t, b1, w2_t, b2):
    devs = jax.devices()
    if len(devs) < 2:
        return _forward_one_core(x, w1_t, b1, w2_t, b2)
    mesh = Mesh(devs[:2], ("b",))
    shard = NamedSharding(mesh, P("b", None))
    repl = NamedSharding(mesh, P(None, None))
    x = jax.lax.with_sharding_constraint(x, shard)
    w1_t = jax.lax.with_sharding_constraint(w1_t, repl)
    b1 = jax.lax.with_sharding_constraint(b1, repl)
    w2_t = jax.lax.with_sharding_constraint(w2_t, repl)
    b2 = jax.lax.with_sharding_constraint(b2, repl)
    fwd = jax.shard_map(
        _forward_one_core, mesh=mesh,
        in_specs=(P("b", None), P(None, None), P(None, None),
                  P(None, None), P(None, None)),
        out_specs=P("b", None), check_vma=False,
    )
    return fwd(x, w1_t, b1, w2_t, b2)
